# Initial kernel scaffold; baseline (speedup 1.0000x reference)
import jax
import jax.numpy as jnp
from jax.experimental import pallas as pl


def kernel(x, rel_pos_embeddings):
    raise NotImplementedError("write your pallas kernel here")


# speedup vs baseline: 8.6820x; 8.6820x over previous
import functools

import jax
import jax.numpy as jnp
from jax import lax
from jax.experimental import pallas as pl
from jax.experimental.pallas import tpu as pltpu
from jax.experimental.pallas import tpu_sc as plsc

_R = 128


@functools.lru_cache(maxsize=None)
def _make_retriever(L, D):
    NC, NS, LANES = 2, 16, 16
    NW = NC * NS
    T = 2 * _R + 1
    q_per_tile = L // NW
    SEG = 4
    K_SEG = L // SEG
    P_LOCAL = K_SEG + q_per_tile
    FIRE = 8

    mesh = plsc.VectorSubcoreMesh(core_axis_name="c", subcore_axis_name="s")

    @functools.partial(
        pl.kernel,
        out_type=jax.ShapeDtypeStruct((L, L, D), jnp.float32),
        mesh=mesh,
        scratch_types=[
            pltpu.VMEM((T, D), jnp.float32),
            pltpu.VMEM((P_LOCAL, D), jnp.float32),
            pltpu.SemaphoreType.DMA,
        ],
    )
    def retrieve(table_hbm, out_hbm, tbl_v, p_v, sem_w):
        cid = lax.axis_index("c")
        sid = lax.axis_index("s")
        wid = sid * NC + cid
        q0 = wid * q_per_tile
        g0 = (L - q_per_tile) - q0

        pltpu.sync_copy(table_hbm, tbl_v)

        for seg in range(SEG):
            k0 = seg * K_SEG

            def build_row(i, _, k0=k0):
                tidx = jnp.clip(g0 + k0 + i - (L - 1 - _R), 0, 2 * _R)
                for col in range(0, D, LANES):
                    p_v[i, pl.ds(col, LANES)] = tbl_v[tidx, pl.ds(col, LANES)]
                return _

            lax.fori_loop(0, P_LOCAL, build_row, 0)

            def fire_drain(t, carry, k0=k0):
                copies = [
                    pltpu.async_copy(
                        p_v.at[pl.ds((q_per_tile - 1) - (t * FIRE + j), K_SEG)],
                        out_hbm.at[q0 + t * FIRE + j].at[pl.ds(k0, K_SEG)],
                        sem_w,
                    )
                    for j in range(FIRE)
                ]
                for cp in copies:
                    cp.wait()
                return carry

            lax.fori_loop(0, q_per_tile // FIRE, fire_drain, 0)

    return retrieve


def kernel(x, rel_pos_embeddings):
    L = x.shape[1]
    D = rel_pos_embeddings.shape[1]
    return _make_retriever(L, D)(rel_pos_embeddings)


# speedup vs baseline: 15.6971x; 1.8080x over previous
import functools

import jax
import jax.numpy as jnp
from jax import lax
from jax.experimental import pallas as pl
from jax.experimental.pallas import tpu as pltpu
from jax.experimental.pallas import tpu_sc as plsc

_R = 128


@functools.lru_cache(maxsize=None)
def _make_retriever(L, D):
    NC, NS, LANES = 2, 16, 16
    NW = NC * NS
    T = 2 * _R + 1
    q_per_tile = L // NW
    SEG = 32
    K_SEG = L // SEG
    CH_SEG = K_SEG * D // 128
    N_PH = 4
    P_BUILD = K_SEG + q_per_tile + N_PH
    CHUNKS = (q_per_tile // N_PH) + CH_SEG

    mesh = plsc.VectorSubcoreMesh(core_axis_name="c", subcore_axis_name="s")

    @functools.partial(
        pl.kernel,
        out_type=jax.ShapeDtypeStruct((L, L * D // 128, 128), jnp.float32),
        mesh=mesh,
        scratch_types=[
            pltpu.VMEM((T, D), jnp.float32),
            pltpu.VMEM((N_PH, CHUNKS, 128), jnp.float32),
            pltpu.SemaphoreType.DMA,
        ],
    )
    def retrieve(table_hbm, out_hbm, tbl_v, pp, sem_w):
        cid = lax.axis_index("c")
        sid = lax.axis_index("s")
        wid = sid * NC + cid
        q0 = wid * q_per_tile
        g0 = (L - q_per_tile) - q0

        pltpu.sync_copy(table_hbm, tbl_v)

        def seg_body(seg, carry):
            gbase = g0 + seg * K_SEG

            def build_g(g, bcarry):
                tidx = jnp.clip(gbase + g - (L - 1 - _R), 0, 2 * _R)
                v0 = tbl_v[tidx, pl.ds(0, LANES)]
                v1 = tbl_v[tidx, pl.ds(LANES, LANES)]
                for phi in range(N_PH):
                    @pl.when(jnp.logical_and(g >= phi, g < phi + N_PH * CHUNKS))
                    def _store(phi=phi):
                        gg = g - phi
                        ch = gg >> 2
                        woff = (gg & 3) * D
                        pp[phi, ch, pl.ds(woff, LANES)] = v0
                        pp[phi, ch, pl.ds(woff + LANES, LANES)] = v1
                return bcarry

            lax.fori_loop(0, P_BUILD, build_g, 0)

            c0 = seg * CH_SEG
            copies = []
            for r in range(q_per_tile):
                o = (q_per_tile - 1) - r
                phi = o & (N_PH - 1)
                j0 = o >> 2
                copies.append(
                    pltpu.async_copy(
                        pp.at[phi].at[pl.ds(j0, CH_SEG), :],
                        out_hbm.at[q0 + r].at[pl.ds(c0, CH_SEG), :],
                        sem_w,
                    )
                )
            for cp in copies:
                cp.wait()
            return carry

        lax.fori_loop(0, SEG, seg_body, 0)

    return retrieve


def kernel(x, rel_pos_embeddings):
    L = x.shape[1]
    D = rel_pos_embeddings.shape[1]
    flat = _make_retriever(L, D)(rel_pos_embeddings)
    return flat.reshape(L, L, D)
